# baseline (speedup 1.0000x reference)
import functools

import jax
import jax.numpy as jnp
import numpy as np
from jax import lax
from jax.experimental import pallas as pl
from jax.experimental.pallas import tpu as pltpu
from jax.sharding import Mesh, PartitionSpec as P

_TIME_CHUNK = 512
_UNROLL = 16


def _rnn_kernel(x_ref, h0_ref, waug_ref, wih_ref, b_ref, wout_ref, bout_ref,
                y_ref, hfin_ref, g_carry, xt_scr, y_scr,
                *, t_total, mask_tail, unroll):
    bb, tc = x_ref.shape
    hid = waug_ref.shape[1]
    aug = waug_ref.shape[0]

    use_mxu = (hid == 256 and bb % 256 == 0)
    cw = 256 if use_mxu else bb // 2
    n_ch = bb // cw

    @pl.when(pl.program_id(0) == 0)
    def _():
        g_carry[...] = jnp.transpose(h0_ref[...])

    xt_scr[...] = jnp.transpose(x_ref[...])

    w_aug = waug_ref[...]
    wih_c = jnp.broadcast_to(wih_ref[...], (hid, cw))
    bias_c = jnp.broadcast_to(b_ref[...], (hid, cw))
    bout_b = jnp.broadcast_to(bout_ref[...], (1, bb))

    t0 = pl.program_id(0) * tc

    def step(t, carry):
        if use_mxu:
            for i, g in enumerate(carry):
                pltpu.matmul_push_rhs(g, staging_register=0, mxu_index=i % 2)
                pltpu.matmul_acc_lhs((i // 2) * 128, w_aug,
                                     mxu_index=i % 2, load_staged_rhs=0)
            pfs = [
                pltpu.matmul_pop((i // 2) * 128, (aug, cw), jnp.float32,
                                 mxu_index=i % 2)
                for i in range(n_ch)
            ]
        else:
            pfs = [jnp.dot(w_aug, g, preferred_element_type=jnp.float32,
                           precision=lax.Precision.HIGHEST) for g in carry]
        y_prev = jnp.concatenate(
            [pf[hid:hid + 1] for pf in pfs], axis=1) + bout_b
        tw = jnp.maximum(t - 1, 0)
        y_scr[pl.ds(tw, 1), :] = y_prev
        x_row = xt_scr[pl.ds(t, 1), :]
        new = []
        for i, (g, pf) in enumerate(zip(carry, pfs)):
            g_new = jnp.tanh(pf[:hid] + wih_c * x_row[:, i * cw:(i + 1) * cw]
                             + bias_c)
            if mask_tail:
                g_new = jnp.where(t0 + t < t_total, g_new, g)
            new.append(g_new)
        return tuple(new)

    g0s = tuple(g_carry[:, i * cw:(i + 1) * cw] for i in range(n_ch))
    gs = lax.fori_loop(0, tc, step, g0s, unroll=unroll)

    wout_c = jnp.broadcast_to(wout_ref[...], (hid, cw))
    y_last = jnp.concatenate(
        [jnp.sum(g * wout_c, axis=0, keepdims=True) for g in gs],
        axis=1) + bout_b
    y_scr[pl.ds(tc - 1, 1), :] = y_last

    y_ref[...] = jnp.transpose(y_scr[...])

    g_fin = jnp.concatenate(gs, axis=1)
    g_carry[...] = g_fin
    hfin_ref[...] = jnp.transpose(g_fin)


def _rnn_forward(x_btf, h_state, weight_ih, weight_hh, bias_ih, bias_hh,
                 out_weight, out_bias):
    B, T, I = x_btf.shape
    H = weight_hh.shape[0]
    assert I == 1 and B % 2 == 0

    h0 = h_state[0].astype(jnp.float32)
    x_bt = x_btf[:, :, 0].astype(jnp.float32)

    w_hh = weight_hh.astype(jnp.float32)
    w_out_row = out_weight.reshape(1, H).astype(jnp.float32)
    w_aug = jnp.concatenate(
        [w_hh, w_out_row, jnp.zeros((7, H), jnp.float32)], axis=0)
    w_ih = weight_ih.reshape(H, 1).astype(jnp.float32)
    bias = (bias_ih + bias_hh).reshape(H, 1).astype(jnp.float32)
    w_out = out_weight.reshape(H, 1).astype(jnp.float32)
    b_out = out_bias.reshape(1, 1).astype(jnp.float32)

    if T <= _TIME_CHUNK:
        tc, t_pad = T, T
    else:
        tc = max(8, (_TIME_CHUNK // 8) * 8)
        t_pad = pl.cdiv(T, tc) * tc
    if t_pad != T:
        x_bt = jnp.pad(x_bt, ((0, 0), (0, t_pad - T)))
    n_chunks = t_pad // tc

    kernel_fn = functools.partial(
        _rnn_kernel, t_total=T, mask_tail=(t_pad != T),
        unroll=min(tc, _UNROLL))

    y_bt, h_fin = pl.pallas_call(
        kernel_fn,
        grid=(n_chunks,),
        in_specs=[
            pl.BlockSpec((B, tc), lambda c: (0, c)),
            pl.BlockSpec((B, H), lambda c: (0, 0)),
            pl.BlockSpec((H + 8, H), lambda c: (0, 0)),
            pl.BlockSpec((H, 1), lambda c: (0, 0)),
            pl.BlockSpec((H, 1), lambda c: (0, 0)),
            pl.BlockSpec((H, 1), lambda c: (0, 0)),
            pl.BlockSpec((1, 1), lambda c: (0, 0)),
        ],
        out_specs=[
            pl.BlockSpec((B, tc), lambda c: (0, c)),
            pl.BlockSpec((B, H), lambda c: (0, 0)),
        ],
        out_shape=[
            jax.ShapeDtypeStruct((B, t_pad), jnp.float32),
            jax.ShapeDtypeStruct((B, H), jnp.float32),
        ],
        scratch_shapes=[
            pltpu.VMEM((H, B), jnp.float32),
            pltpu.VMEM((tc, B), jnp.float32),
            pltpu.VMEM((tc, B), jnp.float32),
        ],
        compiler_params=pltpu.CompilerParams(
            dimension_semantics=("arbitrary",)),
    )(x_bt, h0, w_aug, w_ih, bias, w_out, b_out)

    return y_bt[:, :T, None], h_fin[None]


def kernel(x, h_state, weight_ih, weight_hh, bias_ih, bias_hh,
           out_weight, out_bias):
    args = (x, h_state, weight_ih, weight_hh, bias_ih, bias_hh,
            out_weight, out_bias)
    devs = jax.devices()
    if len(devs) < 2 or x.shape[0] % 2 != 0:
        return _rnn_forward(*args)
    mesh = Mesh(np.array(devs[:2]), ("d",))
    fwd = jax.shard_map(
        _rnn_forward, mesh=mesh,
        in_specs=(P("d"), P(None, "d"), P(), P(), P(), P(), P(), P()),
        out_specs=(P("d"), P(None, "d")),
        check_vma=False)
    return fwd(*args)


# speedup vs baseline: 1.0584x; 1.0584x over previous
import functools

import jax
import jax.numpy as jnp
import numpy as np
from jax import lax
from jax.experimental import pallas as pl
from jax.experimental.pallas import tpu as pltpu
from jax.sharding import Mesh, PartitionSpec as P

_TIME_CHUNK = 512
_UNROLL = 16


def _rnn_kernel(x_ref, h0_ref, waug_ref, wih_ref, b_ref, wout_ref, bout_ref,
                y_ref, hfin_ref, g_carry, xt_scr, y_scr,
                *, t_total, mask_tail, unroll):
    bb, tc = x_ref.shape
    hid = waug_ref.shape[1]
    aug = waug_ref.shape[0]

    use_mxu = (hid == 256 and bb % 256 == 0)
    cw = 256 if use_mxu else bb // 2
    n_ch = bb // cw

    @pl.when(pl.program_id(0) == 0)
    def _():
        g_carry[...] = jnp.transpose(h0_ref[...])

    xt_scr[...] = jnp.transpose(x_ref[...])

    w_aug = waug_ref[...]
    wih_c = jnp.broadcast_to(wih_ref[...], (hid, cw))
    bias_c = jnp.broadcast_to(b_ref[...], (hid, cw))
    bout_b = jnp.broadcast_to(bout_ref[...], (1, bb))

    t0 = pl.program_id(0) * tc

    m_top = (hid // 2) if use_mxu else 0
    m_bot = aug - m_top
    w_top = w_aug[:m_top]
    w_bot = w_aug[m_top:]

    def step(t, carry):
        if use_mxu:
            for i, g in enumerate(carry):
                sr = i % 2
                pltpu.matmul_push_rhs(g, staging_register=sr, mxu_index=0)
                pltpu.matmul_acc_lhs(i * 32, w_top, mxu_index=0,
                                     load_staged_rhs=sr)
                pltpu.matmul_push_rhs(g, staging_register=sr, mxu_index=1)
                pltpu.matmul_acc_lhs(i * 40, w_bot, mxu_index=1,
                                     load_staged_rhs=sr)
            pfs = [
                jnp.concatenate(
                    [pltpu.matmul_pop(i * 32, (m_top, cw), jnp.float32,
                                      mxu_index=0),
                     pltpu.matmul_pop(i * 40, (m_bot, cw), jnp.float32,
                                      mxu_index=1)], axis=0)
                for i in range(n_ch)
            ]
        else:
            pfs = [jnp.dot(w_aug, g, preferred_element_type=jnp.float32,
                           precision=lax.Precision.HIGHEST) for g in carry]
        y_prev = jnp.concatenate(
            [pf[hid:hid + 1] for pf in pfs], axis=1) + bout_b
        tw = jnp.maximum(t - 1, 0)
        y_scr[pl.ds(tw, 1), :] = y_prev
        x_row = xt_scr[pl.ds(t, 1), :]
        new = []
        for i, (g, pf) in enumerate(zip(carry, pfs)):
            g_new = jnp.tanh(pf[:hid] + wih_c * x_row[:, i * cw:(i + 1) * cw]
                             + bias_c)
            if mask_tail:
                g_new = jnp.where(t0 + t < t_total, g_new, g)
            new.append(g_new)
        return tuple(new)

    g0s = tuple(g_carry[:, i * cw:(i + 1) * cw] for i in range(n_ch))
    gs = lax.fori_loop(0, tc, step, g0s, unroll=unroll)

    wout_c = jnp.broadcast_to(wout_ref[...], (hid, cw))
    y_last = jnp.concatenate(
        [jnp.sum(g * wout_c, axis=0, keepdims=True) for g in gs],
        axis=1) + bout_b
    y_scr[pl.ds(tc - 1, 1), :] = y_last

    y_ref[...] = jnp.transpose(y_scr[...])

    g_fin = jnp.concatenate(gs, axis=1)
    g_carry[...] = g_fin
    hfin_ref[...] = jnp.transpose(g_fin)


def _rnn_forward(x_btf, h_state, weight_ih, weight_hh, bias_ih, bias_hh,
                 out_weight, out_bias):
    B, T, I = x_btf.shape
    H = weight_hh.shape[0]
    assert I == 1 and B % 2 == 0

    h0 = h_state[0].astype(jnp.float32)
    x_bt = x_btf[:, :, 0].astype(jnp.float32)

    w_hh = weight_hh.astype(jnp.float32)
    w_out_row = out_weight.reshape(1, H).astype(jnp.float32)
    w_aug = jnp.concatenate(
        [w_hh, w_out_row, jnp.zeros((7, H), jnp.float32)], axis=0)
    w_ih = weight_ih.reshape(H, 1).astype(jnp.float32)
    bias = (bias_ih + bias_hh).reshape(H, 1).astype(jnp.float32)
    w_out = out_weight.reshape(H, 1).astype(jnp.float32)
    b_out = out_bias.reshape(1, 1).astype(jnp.float32)

    if T <= _TIME_CHUNK:
        tc, t_pad = T, T
    else:
        tc = max(8, (_TIME_CHUNK // 8) * 8)
        t_pad = pl.cdiv(T, tc) * tc
    if t_pad != T:
        x_bt = jnp.pad(x_bt, ((0, 0), (0, t_pad - T)))
    n_chunks = t_pad // tc

    kernel_fn = functools.partial(
        _rnn_kernel, t_total=T, mask_tail=(t_pad != T),
        unroll=min(tc, _UNROLL))

    y_bt, h_fin = pl.pallas_call(
        kernel_fn,
        grid=(n_chunks,),
        in_specs=[
            pl.BlockSpec((B, tc), lambda c: (0, c)),
            pl.BlockSpec((B, H), lambda c: (0, 0)),
            pl.BlockSpec((H + 8, H), lambda c: (0, 0)),
            pl.BlockSpec((H, 1), lambda c: (0, 0)),
            pl.BlockSpec((H, 1), lambda c: (0, 0)),
            pl.BlockSpec((H, 1), lambda c: (0, 0)),
            pl.BlockSpec((1, 1), lambda c: (0, 0)),
        ],
        out_specs=[
            pl.BlockSpec((B, tc), lambda c: (0, c)),
            pl.BlockSpec((B, H), lambda c: (0, 0)),
        ],
        out_shape=[
            jax.ShapeDtypeStruct((B, t_pad), jnp.float32),
            jax.ShapeDtypeStruct((B, H), jnp.float32),
        ],
        scratch_shapes=[
            pltpu.VMEM((H, B), jnp.float32),
            pltpu.VMEM((tc, B), jnp.float32),
            pltpu.VMEM((tc, B), jnp.float32),
        ],
        compiler_params=pltpu.CompilerParams(
            dimension_semantics=("arbitrary",)),
    )(x_bt, h0, w_aug, w_ih, bias, w_out, b_out)

    return y_bt[:, :T, None], h_fin[None]


def kernel(x, h_state, weight_ih, weight_hh, bias_ih, bias_hh,
           out_weight, out_bias):
    args = (x, h_state, weight_ih, weight_hh, bias_ih, bias_hh,
            out_weight, out_bias)
    devs = jax.devices()
    if len(devs) < 2 or x.shape[0] % 2 != 0:
        return _rnn_forward(*args)
    mesh = Mesh(np.array(devs[:2]), ("d",))
    fwd = jax.shard_map(
        _rnn_forward, mesh=mesh,
        in_specs=(P("d"), P(None, "d"), P(), P(), P(), P(), P(), P()),
        out_specs=(P("d"), P(None, "d")),
        check_vma=False)
    return fwd(*args)
